# baseline (speedup 1.0000x reference)
import functools
import math

import jax
import jax.numpy as jnp
import numpy as np
from jax import lax
from jax.experimental import pallas as pl
from jax.experimental.pallas import tpu as pltpu
from jax.experimental.pallas import tpu_sc as plsc

_MAX_LEN = 1500
_NC = 2
_NS = 16
_SPLIT = 104
_CHUNK = 32


def _pe_np(max_len: int, d: int) -> np.ndarray:
    position = np.arange(0, max_len, dtype=np.float32)[:, None]
    div_term = np.exp(
        np.arange(0, d, 2, dtype=np.float32) * -(math.log(10000.0) / d)
    )
    pe = np.zeros((max_len, d), dtype=np.float32)
    pe[:, 0::2] = np.sin(position * div_term)
    pe[:, 1::2] = np.cos(position * div_term)
    return pe


def _sc_kernel(idx_flat, table, pe3, bsz, l):
    v, d = table.shape
    la, lb = _SPLIT, l - _SPLIT
    bpw_a = 8 * ((la * v // _NS + 7) // 8)
    bpw_b = lb * v // _NS
    half_pad = _NS * bpw_a
    per_w = bsz // _NS
    nb = 4
    n4 = per_w // nb
    nch = per_w // _CHUNK
    cw = _CHUNK * l
    mesh = plsc.VectorSubcoreMesh(core_axis_name="c", subcore_axis_name="s")

    @functools.partial(
        pl.kernel,
        mesh=mesh,
        out_type=jax.ShapeDtypeStruct((bsz, l, d), jnp.float32),
        scratch_types=[
            pltpu.VMEM((per_w * la + 16,), jnp.int32),
            pltpu.VMEM((2 * cw,), jnp.int32),
            pltpu.VMEM((_CHUNK, d), jnp.float32),
            pltpu.VMEM((8, 1, d), jnp.float32),
            pltpu.VMEM((v, d), jnp.float32),
            pltpu.VMEM((nb, la, d), jnp.float32),
            pltpu.VMEM_SHARED((half_pad, d), jnp.float32),
        ]
        + [pltpu.SemaphoreType.DMA] * (2 * nb + 1),
    )
    def k(idx_hbm, tab_hbm, pe_hbm, out_hbm, flat_v, ichunk, build_v,
          pe_t, tab_t, rows_v, fused_sp, *sems):
        gsems, wsems, isem = sems[:nb], sems[nb:2 * nb], sems[2 * nb]
        cid = lax.axis_index("c")
        sid = lax.axis_index("s")
        b0 = sid * per_w
        iota = lax.iota(jnp.int32, 16)

        def ichunk_load(c, parity):
            return pltpu.make_async_copy(
                idx_hbm.at[pl.ds((b0 + c * _CHUNK) * l, cw)],
                ichunk.at[pl.ds(parity * cw, cw)],
                isem,
            )

        def run(n_pos, col0, pos0, bpw, chunks):
            ichunk_load(0, 0).start()
            p_lo = (sid * bpw) // v
            pltpu.sync_copy(pe_hbm.at[pl.ds(pos0 + p_lo, 8)], pe_t)
            pltpu.sync_copy(tab_hbm, tab_t)

            row_base = sid * bpw
            done = 0
            for csz in chunks:
                coff = row_base + done

                def brow(r, carry, coff=coff):
                    rr = coff + r
                    p = rr // v - p_lo
                    vv = lax.rem(rr, v)
                    for q in range(d // 16):
                        build_v[r, pl.ds(q * 16, 16)] = (
                            pe_t[p, 0, pl.ds(q * 16, 16)]
                            + tab_t[vv, pl.ds(q * 16, 16)]
                        )
                    return carry

                lax.fori_loop(0, csz, brow, 0)
                pltpu.sync_copy(
                    build_v.at[pl.ds(0, csz)],
                    fused_sp.at[pl.ds(coff, csz)],
                )
                done += csz

            def flatten(c):
                ib = lax.rem(c, 2) * cw

                def row(r, carry):
                    src = ib + r * l + col0
                    dst = (c * _CHUNK + r) * n_pos
                    for t in range((n_pos + 15) // 16):
                        x = ichunk[pl.ds(src + 16 * t, 16)]
                        flat_v[pl.ds(dst + 16 * t, 16)] = (
                            x + (iota + 16 * t) * v
                        )
                    return carry

                lax.fori_loop(0, _CHUNK, row, 0)

            ichunk_load(0, 0).wait()
            flatten(0)
            ichunk_load(1, 1).start()
            plsc.subcore_barrier()

            def gather(i, b):
                return pltpu.make_async_copy(
                    fused_sp.at[flat_v.at[pl.ds(i * n_pos, n_pos)]],
                    rows_v.at[b, pl.ds(0, n_pos)],
                    gsems[b],
                )

            def write(i, b):
                return pltpu.make_async_copy(
                    rows_v.at[b, pl.ds(0, n_pos)],
                    out_hbm.at[b0 + i, pl.ds(pos0, n_pos)],
                    wsems[b],
                )

            for b in range(nb - 1):
                gather(b, b).start()

            def body(i4, carry):
                @pl.when(lax.rem(i4, _CHUNK // nb) == 0)
                def _():
                    c1 = i4 // (_CHUNK // nb) + 1

                    @pl.when(c1 < nch)
                    def _():
                        ichunk_load(c1, lax.rem(c1, 2)).wait()
                        flatten(c1)

                        @pl.when(c1 + 1 < nch)
                        def _():
                            ichunk_load(c1 + 1, lax.rem(c1 + 1, 2)).start()

                for b in range(nb):
                    i = i4 * nb + b
                    gather(i, b).wait()
                    write(i, b).start()
                    bm1 = (b - 1) % nb
                    if b == 0:

                        @pl.when(i4 > 0)
                        def _():
                            write(i - 1, bm1).wait()

                        gather(i + nb - 1, bm1).start()
                    else:
                        write(i - 1, bm1).wait()

                        @pl.when(i4 < n4 - 1)
                        def _():
                            gather(i + nb - 1, bm1).start()

                return carry

            lax.fori_loop(0, n4, body, 0)
            write(per_w - 1, (per_w - 1) % nb).wait()

        @pl.when(cid == 0)
        def _():
            run(la, 0, 0, bpw_a, (32, 32, 32, 32, 32, 24))

        @pl.when(cid == 1)
        def _():
            run(lb, _SPLIT, _SPLIT, bpw_b, (32, 32, 32, 32, 32, 8))

    return k(idx_flat, table, pe3)


def kernel(indices, table):
    b, l = indices.shape
    v, d = table.shape
    pe = jnp.asarray(_pe_np(_MAX_LEN, d)[: l + 8])
    return _sc_kernel(
        indices.reshape(b * l),
        table,
        pe.reshape(l + 8, 1, d),
        b,
        l,
    )


# speedup vs baseline: 1.0120x; 1.0120x over previous
import functools
import math

import jax
import jax.numpy as jnp
import numpy as np
from jax import lax
from jax.experimental import pallas as pl
from jax.experimental.pallas import tpu as pltpu
from jax.experimental.pallas import tpu_sc as plsc

_MAX_LEN = 1500
_NC = 2
_NS = 16
_SPLIT = 104
_CHUNK = 32


def _pe_np(max_len: int, d: int) -> np.ndarray:
    position = np.arange(0, max_len, dtype=np.float32)[:, None]
    div_term = np.exp(
        np.arange(0, d, 2, dtype=np.float32) * -(math.log(10000.0) / d)
    )
    pe = np.zeros((max_len, d), dtype=np.float32)
    pe[:, 0::2] = np.sin(position * div_term)
    pe[:, 1::2] = np.cos(position * div_term)
    return pe


def _sc_kernel(idx_flat, table, pe3, bsz, l):
    v, d = table.shape
    la, lb = _SPLIT, l - _SPLIT
    bpw_a = 8 * ((la * v // _NS + 7) // 8)
    bpw_b = lb * v // _NS
    half_pad = _NS * bpw_a
    per_w = bsz // _NS
    nb = 4
    n4 = per_w // nb
    nch = per_w // _CHUNK
    cw = _CHUNK * l
    mesh = plsc.VectorSubcoreMesh(core_axis_name="c", subcore_axis_name="s")

    @functools.partial(
        pl.kernel,
        mesh=mesh,
        out_type=jax.ShapeDtypeStruct((bsz, l, d), jnp.float32),
        scratch_types=[
            pltpu.VMEM((per_w * la + 16,), jnp.int32),
            pltpu.VMEM((2 * cw,), jnp.int32),
            pltpu.VMEM((_CHUNK, d), jnp.float32),
            pltpu.VMEM((8, 1, d), jnp.float32),
            pltpu.VMEM((v, d), jnp.float32),
            pltpu.VMEM((nb, la, d), jnp.float32),
            pltpu.VMEM_SHARED((half_pad, d), jnp.float32),
        ]
        + [pltpu.SemaphoreType.DMA] * (2 * nb + 1),
    )
    def k(idx_hbm, tab_hbm, pe_hbm, out_hbm, flat_v, ichunk, build_v,
          pe_t, tab_t, rows_v, fused_sp, *sems):
        gsems, wsems, isem = sems[:nb], sems[nb:2 * nb], sems[2 * nb]
        cid = lax.axis_index("c")
        sid = lax.axis_index("s")
        b0 = sid * per_w
        iota = lax.iota(jnp.int32, 16)

        def ichunk_load(c, parity):
            return pltpu.make_async_copy(
                idx_hbm.at[pl.ds((b0 + c * _CHUNK) * l, cw)],
                ichunk.at[pl.ds(parity * cw, cw)],
                isem,
            )

        def run(n_pos, col0, pos0, bpw, chunks):
            ichunk_load(0, 0).start()
            p_lo = (sid * bpw) // v
            pltpu.sync_copy(pe_hbm.at[pl.ds(pos0 + p_lo, 8)], pe_t)
            pltpu.sync_copy(tab_hbm, tab_t)

            row_base = sid * bpw
            done = 0
            for csz in chunks:
                coff = row_base + done

                def brow(r, carry, coff=coff):
                    rr = coff + r
                    p = rr // v - p_lo
                    vv = lax.rem(rr, v)
                    for q in range(d // 16):
                        build_v[r, pl.ds(q * 16, 16)] = (
                            pe_t[p, 0, pl.ds(q * 16, 16)]
                            + tab_t[vv, pl.ds(q * 16, 16)]
                        )
                    return carry

                lax.fori_loop(0, csz, brow, 0)
                pltpu.sync_copy(
                    build_v.at[pl.ds(0, csz)],
                    fused_sp.at[pl.ds(coff, csz)],
                )
                done += csz

            def flatten(c):
                ib = lax.rem(c, 2) * cw

                def row(r, carry):
                    src = ib + r * l + col0
                    dst = (c * _CHUNK + r) * n_pos
                    for t in range((n_pos + 15) // 16):
                        x = ichunk[pl.ds(src + 16 * t, 16)]
                        flat_v[pl.ds(dst + 16 * t, 16)] = (
                            x + (iota + 16 * t) * v
                        )
                    return carry

                lax.fori_loop(0, _CHUNK, row, 0)

            ichunk_load(0, 0).wait()
            flatten(0)
            ichunk_load(1, 1).start()
            plsc.subcore_barrier()

            def gather(i, b):
                return pltpu.make_async_copy(
                    fused_sp.at[flat_v.at[pl.ds(i * n_pos, n_pos)]],
                    rows_v.at[b, pl.ds(0, n_pos)],
                    gsems[b],
                )

            def write(i, b):
                return pltpu.make_async_copy(
                    rows_v.at[b, pl.ds(0, n_pos)],
                    out_hbm.at[b0 + i, pl.ds(pos0, n_pos)],
                    wsems[b],
                )

            for b in range(nb - 1):
                gather(b, b).start()

            def body(i4, carry):
                @pl.when(lax.rem(i4, _CHUNK // nb) == 0)
                def _():
                    c1 = i4 // (_CHUNK // nb) + 1

                    @pl.when(c1 < nch)
                    def _():
                        ichunk_load(c1, lax.rem(c1, 2)).wait()

                        @pl.when(c1 + 1 < nch)
                        def _():
                            ichunk_load(c1 + 1, lax.rem(c1 + 1, 2)).start()

                @pl.when(i4 + _CHUNK // nb < n4)
                def _():
                    j4 = i4 + _CHUNK // nb
                    par = lax.rem(j4 // (_CHUNK // nb), 2) * cw
                    rl = lax.rem(j4, _CHUNK // nb) * nb
                    for b in range(nb):
                        src = par + (rl + b) * l + col0
                        dst = (j4 * nb + b) * n_pos
                        for t in range((n_pos + 15) // 16):
                            x = ichunk[pl.ds(src + 16 * t, 16)]
                            flat_v[pl.ds(dst + 16 * t, 16)] = (
                                x + (iota + 16 * t) * v
                            )

                for b in range(nb):
                    i = i4 * nb + b
                    gather(i, b).wait()
                    write(i, b).start()
                    bm1 = (b - 1) % nb
                    if b == 0:

                        @pl.when(i4 > 0)
                        def _():
                            write(i - 1, bm1).wait()

                        gather(i + nb - 1, bm1).start()
                    else:
                        write(i - 1, bm1).wait()

                        @pl.when(i4 < n4 - 1)
                        def _():
                            gather(i + nb - 1, bm1).start()

                return carry

            lax.fori_loop(0, n4, body, 0)
            write(per_w - 1, (per_w - 1) % nb).wait()

        @pl.when(cid == 0)
        def _():
            run(la, 0, 0, bpw_a, (32, 32, 32, 32, 32, 24))

        @pl.when(cid == 1)
        def _():
            run(lb, _SPLIT, _SPLIT, bpw_b, (32, 32, 32, 32, 32, 8))

    return k(idx_flat, table, pe3)


def kernel(indices, table):
    b, l = indices.shape
    v, d = table.shape
    pe = jnp.asarray(_pe_np(_MAX_LEN, d)[: l + 8])
    return _sc_kernel(
        indices.reshape(b * l),
        table,
        pe.reshape(l + 8, 1, d),
        b,
        l,
    )


# speedup vs baseline: 1.0456x; 1.0332x over previous
import functools
import math

import jax
import jax.numpy as jnp
import numpy as np
from jax import lax
from jax.experimental import pallas as pl
from jax.experimental.pallas import tpu as pltpu
from jax.experimental.pallas import tpu_sc as plsc

_MAX_LEN = 1500
_NC = 2
_NS = 16
_SPLIT = 104
_CHUNK = 32


def _pe_np(max_len: int, d: int) -> np.ndarray:
    position = np.arange(0, max_len, dtype=np.float32)[:, None]
    div_term = np.exp(
        np.arange(0, d, 2, dtype=np.float32) * -(math.log(10000.0) / d)
    )
    pe = np.zeros((max_len, d), dtype=np.float32)
    pe[:, 0::2] = np.sin(position * div_term)
    pe[:, 1::2] = np.cos(position * div_term)
    return pe


def _sc_kernel(idx_flat, table, pe3, bsz, l):
    v, d = table.shape
    la, lb = _SPLIT, l - _SPLIT
    half_rows = la * v
    per_w = bsz // _NS
    nb = 4
    n4 = per_w // nb
    nch = per_w // _CHUNK
    cw = _CHUNK * l
    mesh = plsc.VectorSubcoreMesh(core_axis_name="c", subcore_axis_name="s")

    @functools.partial(
        pl.kernel,
        mesh=mesh,
        out_type=jax.ShapeDtypeStruct((bsz, l, d), jnp.float32),
        scratch_types=[
            pltpu.VMEM((per_w * la + 16,), jnp.int32),
            pltpu.VMEM((2 * cw,), jnp.int32),
            pltpu.VMEM((2 * v, d), jnp.float32),
            pltpu.VMEM((8, 1, d), jnp.float32),
            pltpu.VMEM((v, d), jnp.float32),
            pltpu.VMEM((nb, la, d), jnp.float32),
            pltpu.VMEM_SHARED((half_rows, d), jnp.float32),
        ]
        + [pltpu.SemaphoreType.DMA] * (2 * nb + 1),
    )
    def k(idx_hbm, tab_hbm, pe_hbm, out_hbm, flat_v, ichunk, build_v,
          pe_t, tab_t, rows_v, fused_sp, *sems):
        gsems, wsems, isem = sems[:nb], sems[nb:2 * nb], sems[2 * nb]
        cid = lax.axis_index("c")
        sid = lax.axis_index("s")
        b0 = sid * per_w
        iota = lax.iota(jnp.int32, 16)

        def ichunk_load(c, parity):
            return pltpu.make_async_copy(
                idx_hbm.at[pl.ds((b0 + c * _CHUNK) * l, cw)],
                ichunk.at[pl.ds(parity * cw, cw)],
                isem,
            )

        def run(n_pos, col0, pos0, npb):
            ichunk_load(0, 0).start()

            @pl.when(sid * npb < n_pos)
            def _():
                pltpu.sync_copy(pe_hbm.at[pl.ds(pos0 + sid * npb, 8)], pe_t)
                pltpu.sync_copy(tab_hbm, tab_t)

                for jj in range(npb // 2):
                    for k2 in range(2):
                        pev = [
                            pe_t[jj * 2 + k2, 0, pl.ds(q * 16, 16)]
                            for q in range(d // 16)
                        ]

                        def brow(vv, carry, pev=pev, k2=k2):
                            for q in range(d // 16):
                                build_v[k2 * v + vv, pl.ds(q * 16, 16)] = (
                                    pev[q] + tab_t[vv, pl.ds(q * 16, 16)]
                                )
                            return carry

                        lax.fori_loop(0, v, brow, 0)
                    pltpu.sync_copy(
                        build_v,
                        fused_sp.at[pl.ds((sid * npb + jj * 2) * v, 2 * v)],
                    )

            def flatten(c):
                ib = lax.rem(c, 2) * cw

                def row(r, carry):
                    src = ib + r * l + col0
                    dst = (c * _CHUNK + r) * n_pos
                    for t in range((n_pos + 15) // 16):
                        x = ichunk[pl.ds(src + 16 * t, 16)]
                        flat_v[pl.ds(dst + 16 * t, 16)] = (
                            x + (iota + 16 * t) * v
                        )
                    return carry

                lax.fori_loop(0, _CHUNK, row, 0)

            ichunk_load(0, 0).wait()
            flatten(0)
            ichunk_load(1, 1).start()
            plsc.subcore_barrier()

            def gather(i, b):
                return pltpu.make_async_copy(
                    fused_sp.at[flat_v.at[pl.ds(i * n_pos, n_pos)]],
                    rows_v.at[b, pl.ds(0, n_pos)],
                    gsems[b],
                )

            def write(i, b):
                return pltpu.make_async_copy(
                    rows_v.at[b, pl.ds(0, n_pos)],
                    out_hbm.at[b0 + i, pl.ds(pos0, n_pos)],
                    wsems[b],
                )

            for b in range(nb - 1):
                gather(b, b).start()

            def body(i4, carry):
                @pl.when(lax.rem(i4, _CHUNK // nb) == 0)
                def _():
                    c1 = i4 // (_CHUNK // nb) + 1

                    @pl.when(c1 < nch)
                    def _():
                        ichunk_load(c1, lax.rem(c1, 2)).wait()

                        @pl.when(c1 + 1 < nch)
                        def _():
                            ichunk_load(c1 + 1, lax.rem(c1 + 1, 2)).start()

                @pl.when(i4 + _CHUNK // nb < n4)
                def _():
                    j4 = i4 + _CHUNK // nb
                    par = lax.rem(j4 // (_CHUNK // nb), 2) * cw
                    rl = lax.rem(j4, _CHUNK // nb) * nb
                    for b in range(nb):
                        src = par + (rl + b) * l + col0
                        dst = (j4 * nb + b) * n_pos
                        for t in range((n_pos + 15) // 16):
                            x = ichunk[pl.ds(src + 16 * t, 16)]
                            flat_v[pl.ds(dst + 16 * t, 16)] = (
                                x + (iota + 16 * t) * v
                            )

                for b in range(nb):
                    i = i4 * nb + b
                    gather(i, b).wait()
                    write(i, b).start()
                    bm1 = (b - 1) % nb
                    if b == 0:

                        @pl.when(i4 > 0)
                        def _():
                            write(i - 1, bm1).wait()

                        gather(i + nb - 1, bm1).start()
                    else:
                        write(i - 1, bm1).wait()

                        @pl.when(i4 < n4 - 1)
                        def _():
                            gather(i + nb - 1, bm1).start()

                return carry

            lax.fori_loop(0, n4, body, 0)
            write(per_w - 1, (per_w - 1) % nb).wait()

        @pl.when(cid == 0)
        def _():
            run(la, 0, 0, 8)

        @pl.when(cid == 1)
        def _():
            run(lb, _SPLIT, _SPLIT, 6)

    return k(idx_flat, table, pe3)


def kernel(indices, table):
    b, l = indices.shape
    v, d = table.shape
    pe = jnp.asarray(_pe_np(_MAX_LEN, d)[: l + 8])
    return _sc_kernel(
        indices.reshape(b * l),
        table,
        pe.reshape(l + 8, 1, d),
        b,
        l,
    )
